# baseline (speedup 1.0000x reference)
import functools

import jax
import jax.numpy as jnp
from jax import lax
from jax.experimental import pallas as pl
from jax.experimental.pallas import tpu as pltpu
from jax.experimental.pallas import tpu_sc as plsc

_NC = 2
_NS = 16
_L = 16
_NW = _NC * _NS


@functools.lru_cache(maxsize=None)
def _build_sc_kernel(B: int, S: int, D: int, interpret: bool = False):
    rows_total = B * S
    rpw = rows_total // _NW
    wpb = _NW // B
    seg = S // wpb
    nvec_row = S // _L
    nvec_seg = seg // _L
    CH = 64

    mesh = plsc.VectorSubcoreMesh(
        core_axis_name="c", subcore_axis_name="s",
        num_cores=_NC, num_subcores=_NS,
    )

    @functools.partial(
        pl.kernel,
        out_type=jax.ShapeDtypeStruct((rows_total, D), jnp.float32),
        mesh=mesh,
        scratch_types=[
            pltpu.VMEM((S,), jnp.int32),
            pltpu.VMEM((rpw,), jnp.int32),
            pltpu.VMEM((rpw,), jnp.float32),
            pltpu.VMEM((CH, D), jnp.float32),
            pltpu.SemaphoreType.DMA,
        ],
        compiler_params=pltpu.CompilerParams(needs_layout_passes=False),
        interpret=interpret,
    )
    def sc_kernel(mask_hbm, w_hbm, out_hbm, maskv, idxv, mval, rows, sem):
        wid = lax.axis_index("s") * _NC + lax.axis_index("c")
        b = wid // wpb
        chunk = wid % wpb
        nvec_pref = chunk * nvec_seg

        pltpu.sync_copy(mask_hbm.at[b], maskv)

        def pref_body(j, acc):
            v = maskv[pl.ds(j * _L, _L)]
            return acc + jnp.where(j < nvec_pref, jnp.sum(v), 0)

        base = lax.fori_loop(0, nvec_row, pref_body, jnp.int32(0))

        def seg_body(j, carry):
            v = maskv[pl.ds((nvec_pref + j) * _L, _L)]
            inc = plsc.cumsum(v) + carry
            idxv[pl.ds(j * _L, _L)] = jnp.maximum(inc - 1, 0)
            mval[pl.ds(j * _L, _L)] = jnp.where(v != 0, 1.0, 0.0).astype(jnp.float32)
            return carry + jnp.sum(v)

        lax.fori_loop(0, nvec_seg, seg_body, base)

        for t in range(rpw // CH):
            pltpu.async_copy(
                w_hbm.at[idxv.at[pl.ds(t * CH, CH)]], rows, sem
            ).wait()

            def row_body(r, _):
                m = plsc.load_gather(
                    mval, [jnp.full((_L,), t * CH, jnp.int32) + r]
                )

                def vec_body(k, __):
                    rows[r, pl.ds(k * _L, _L)] = rows[r, pl.ds(k * _L, _L)] * m
                    return 0

                lax.fori_loop(0, D // _L, vec_body, 0)
                return 0

            lax.fori_loop(0, CH, row_body, 0)
            pltpu.sync_copy(rows, out_hbm.at[pl.ds(wid * rpw + t * CH, CH)])

    return sc_kernel


def kernel(tokens, past_kv_pos_offset, attention_mask, W_pos):
    B, S = attention_mask.shape
    D = W_pos.shape[-1]
    mask = attention_mask.astype(jnp.int32)
    out = _build_sc_kernel(B, S, D)(mask, W_pos.astype(jnp.float32))
    return out.reshape(B, S, D)


# speedup vs baseline: 1.9340x; 1.9340x over previous
import functools

import jax
import jax.numpy as jnp
from jax import lax
from jax.experimental import pallas as pl
from jax.experimental.pallas import tpu as pltpu
from jax.experimental.pallas import tpu_sc as plsc

_NC = 2
_NS = 16
_L = 16
_NW = _NC * _NS


@functools.lru_cache(maxsize=None)
def _build_sc_kernel(B: int, S: int, D: int, interpret: bool = False):
    rows_total = B * S
    rpw = rows_total // _NW
    wpb = _NW // B
    seg = S // wpb
    nvec_row = S // _L
    nvec_seg = seg // _L
    CH = 64

    mesh = plsc.VectorSubcoreMesh(
        core_axis_name="c", subcore_axis_name="s",
        num_cores=_NC, num_subcores=_NS,
    )

    @functools.partial(
        pl.kernel,
        out_type=jax.ShapeDtypeStruct((rows_total, D), jnp.float32),
        mesh=mesh,
        scratch_types=[
            pltpu.VMEM((S,), jnp.int32),
            pltpu.VMEM((rpw,), jnp.int32),
            pltpu.VMEM((rpw,), jnp.float32),
            pltpu.VMEM((CH, D), jnp.float32),
            pltpu.SemaphoreType.DMA,
        ],
        compiler_params=pltpu.CompilerParams(needs_layout_passes=False),
        interpret=interpret,
    )
    def sc_kernel(mask_hbm, w_hbm, out_hbm, maskv, idxv, mval, rows, sem):
        wid = lax.axis_index("s") * _NC + lax.axis_index("c")
        b = wid // wpb
        chunk = wid % wpb
        nvec_pref = chunk * nvec_seg

        pltpu.sync_copy(mask_hbm.at[b], maskv)

        def pref_body(j, acc):
            v = maskv[pl.ds(j * _L, _L)]
            return acc + jnp.where(j < nvec_pref, v, 0)

        zero_v = jnp.zeros((_L,), jnp.int32)
        base = jnp.sum(lax.fori_loop(0, nvec_row, pref_body, zero_v))

        def seg_body(j, carry):
            v = maskv[pl.ds((nvec_pref + j) * _L, _L)]
            inc = plsc.cumsum(v) + carry
            idxv[pl.ds(j * _L, _L)] = jnp.maximum(inc - 1, 0)
            mval[pl.ds(j * _L, _L)] = jnp.where(v != 0, 1.0, 0.0).astype(jnp.float32)
            return carry + jnp.sum(v)

        lax.fori_loop(0, nvec_seg, seg_body, base)

        for t in range(rpw // CH):
            pltpu.async_copy(
                w_hbm.at[idxv.at[pl.ds(t * CH, CH)]], rows, sem
            ).wait()

            def row_body(r, _):
                m = plsc.load_gather(
                    mval, [jnp.full((_L,), t * CH, jnp.int32) + r]
                )
                for k in range(D // _L):
                    rows[r, pl.ds(k * _L, _L)] = rows[r, pl.ds(k * _L, _L)] * m
                return 0

            lax.fori_loop(0, CH, row_body, 0)
            pltpu.sync_copy(rows, out_hbm.at[pl.ds(wid * rpw + t * CH, CH)])

    return sc_kernel


def kernel(tokens, past_kv_pos_offset, attention_mask, W_pos):
    B, S = attention_mask.shape
    D = W_pos.shape[-1]
    mask = attention_mask.astype(jnp.int32)
    out = _build_sc_kernel(B, S, D)(mask, W_pos.astype(jnp.float32))
    return out.reshape(B, S, D)
